# baseline (speedup 1.0000x reference)
import functools

import jax
import jax.numpy as jnp
from jax import lax
from jax.experimental import pallas as pl
from jax.experimental.pallas import tpu as pltpu
from jax.experimental.pallas import tpu_sc as plsc

IMG = 384
PATCH = 16
FEAT = IMG // PATCH
NUM_PATCHES = FEAT * FEAT
CUTOFF = NUM_PATCHES // 2
NHEADS = 12
BATCH = 16
BB = 4
LANES = 16
NCH = NUM_PATCHES // LANES

_I32_MIN = -(2 ** 31)
_I32_MAXP = (1 << 31) - 1


def _sc_mask_body(att_hbm, out_hbm, rows_v, vals_v, skey_v, mask_v):
    c = lax.axis_index("c")
    s = lax.axis_index("s")
    wid = s * 2 + c

    @pl.when(wid < BATCH)
    def _():
        pltpu.sync_copy(att_hbm.at[wid], rows_v)

        def chunk_sum(i, carry):
            def hsum(h, acc):
                return acc + rows_v[pl.ds(h * NUM_PATCHES + i * LANES, LANES)]
            acc = lax.fori_loop(0, NHEADS, hsum, jnp.zeros((LANES,), jnp.float32))
            vals_v[pl.ds(i * LANES, LANES)] = acc
            bits = lax.bitcast_convert_type(acc, jnp.int32)
            skey_v[pl.ds(i * LANES, LANES)] = jnp.where(
                bits >= 0, bits, bits ^ jnp.int32(_I32_MAXP))
            return carry

        lax.fori_loop(0, NCH, chunk_sum, 0)

        def search_bit(j, tu):
            bp = lax.shift_left(jnp.full((LANES,), 1, jnp.int32),
                                jnp.int32(31) - j)
            cand_u = tu | bp
            cand_s = cand_u ^ jnp.int32(_I32_MIN)

            def cnt_chunk(i, cnt):
                sk = skey_v[pl.ds(i * LANES, LANES)]
                return cnt + plsc.all_reduce_population_count(sk >= cand_s)

            cnt = lax.fori_loop(0, NCH, cnt_chunk,
                                jnp.zeros((LANES,), jnp.int32))
            return jnp.where(cnt >= CUTOFF, cand_u, tu)

        tu = lax.fori_loop(0, 32, search_bit, jnp.zeros((LANES,), jnp.int32))
        ts = tu ^ jnp.int32(_I32_MIN)
        thr_bits = jnp.where(ts >= 0, ts, ts ^ jnp.int32(_I32_MAXP))
        thr = lax.bitcast_convert_type(thr_bits, jnp.float32)

        def mask_chunk(i, carry):
            v = vals_v[pl.ds(i * LANES, LANES)]
            mask_v[pl.ds(i * LANES, LANES)] = jnp.where(
                v > thr, jnp.float32(1), jnp.float32(0))
            return carry

        lax.fori_loop(0, NCH, mask_chunk, 0)
        pltpu.sync_copy(mask_v, out_hbm.at[wid])


_sc_mask = functools.partial(
    pl.kernel,
    out_type=jax.ShapeDtypeStruct((BATCH, NUM_PATCHES), jnp.float32),
    mesh=plsc.VectorSubcoreMesh(core_axis_name="c", subcore_axis_name="s"),
    compiler_params=pltpu.CompilerParams(needs_layout_passes=False),
    scratch_types=[
        pltpu.VMEM((NHEADS * NUM_PATCHES,), jnp.float32),
        pltpu.VMEM((NUM_PATCHES,), jnp.float32),
        pltpu.VMEM((NUM_PATCHES,), jnp.int32),
        pltpu.VMEM((NUM_PATCHES,), jnp.float32),
    ],
)(_sc_mask_body)


def _tc_body(mask_ref, img_ref, out_ref):
    b = pl.program_id(0)
    p_cols = jnp.where(
        jax.lax.broadcasted_iota(jnp.int32, (FEAT, IMG), 1) // PATCH
        == jax.lax.broadcasted_iota(jnp.int32, (FEAT, IMG), 0),
        1.0, 0.0).astype(jnp.float32)
    p_rows = jnp.where(
        jax.lax.broadcasted_iota(jnp.int32, (IMG, FEAT), 0) // PATCH
        == jax.lax.broadcasted_iota(jnp.int32, (IMG, FEAT), 1),
        1.0, 0.0).astype(jnp.float32)
    for j in range(BB):
        m = mask_ref[BB * b + j]
        mp = jax.lax.dot_general(m, p_cols, (((1,), (0,)), ((), ())),
                                 preferred_element_type=jnp.float32)
        m_full = jax.lax.dot_general(p_rows, mp, (((1,), (0,)), ((), ())),
                                     preferred_element_type=jnp.float32)
        out_ref[j] = img_ref[j] * m_full[None, :, :]


def kernel(x, input_images):
    B, NH = x.shape[0], x.shape[1]
    att = x[:, :, 0, 1:].reshape(B, NH * NUM_PATCHES)
    mask = _sc_mask(att)
    mask24 = mask.reshape(B, FEAT, FEAT)
    return pl.pallas_call(
        _tc_body,
        grid=(B // BB,),
        in_specs=[
            pl.BlockSpec((B, FEAT, FEAT), lambda b: (0, 0, 0)),
            pl.BlockSpec((BB, 3, IMG, IMG), lambda b: (b, 0, 0, 0)),
        ],
        out_specs=pl.BlockSpec((BB, 3, IMG, IMG), lambda b: (b, 0, 0, 0)),
        out_shape=jax.ShapeDtypeStruct(input_images.shape, input_images.dtype),
    )(mask24, input_images)


# speedup vs baseline: 1.0938x; 1.0938x over previous
import functools

import jax
import jax.numpy as jnp
from jax import lax
from jax.experimental import pallas as pl
from jax.experimental.pallas import tpu as pltpu
from jax.experimental.pallas import tpu_sc as plsc

IMG = 384
PATCH = 16
FEAT = IMG // PATCH
NUM_PATCHES = FEAT * FEAT
CUTOFF = NUM_PATCHES // 2
NHEADS = 12
BATCH = 16
BB = 4
LANES = 16
NCH = NUM_PATCHES // LANES

_I32_MIN = -(2 ** 31)
_I32_MAXP = (1 << 31) - 1


def _sc_mask_body(att_hbm, out_hbm, rows_v, vals_v, skey_v, mask_v):
    wid = lax.axis_index("s")

    pltpu.sync_copy(att_hbm.at[wid], rows_v)

    for i in range(NCH):
        acc = rows_v[pl.ds(i * LANES, LANES)]
        for h in range(1, NHEADS):
            acc = acc + rows_v[pl.ds(h * NUM_PATCHES + i * LANES, LANES)]
        vals_v[pl.ds(i * LANES, LANES)] = acc
        bits = lax.bitcast_convert_type(acc, jnp.int32)
        skey_v[pl.ds(i * LANES, LANES)] = jnp.where(
            bits >= 0, bits, bits ^ jnp.int32(_I32_MAXP))

    def search_bit(j, tu):
        bp = lax.shift_left(jnp.full((LANES,), 1, jnp.int32),
                            jnp.int32(31) - j)
        cand_u = tu | bp
        cand_s = cand_u ^ jnp.int32(_I32_MIN)
        cnt = jnp.zeros((LANES,), jnp.int32)
        for i in range(NCH):
            sk = skey_v[pl.ds(i * LANES, LANES)]
            cnt = cnt + plsc.all_reduce_population_count(sk >= cand_s)
        return jnp.where(cnt >= CUTOFF, cand_u, tu)

    tu = lax.fori_loop(0, 32, search_bit, jnp.zeros((LANES,), jnp.int32))
    ts = tu ^ jnp.int32(_I32_MIN)
    thr_bits = jnp.where(ts >= 0, ts, ts ^ jnp.int32(_I32_MAXP))
    thr = lax.bitcast_convert_type(thr_bits, jnp.float32)

    for i in range(NCH):
        v = vals_v[pl.ds(i * LANES, LANES)]
        mask_v[pl.ds(i * LANES, LANES)] = jnp.where(
            v > thr, jnp.float32(1), jnp.float32(0))

    pltpu.sync_copy(mask_v, out_hbm.at[wid])


_sc_mask = functools.partial(
    pl.kernel,
    out_type=jax.ShapeDtypeStruct((BATCH, NUM_PATCHES), jnp.float32),
    mesh=plsc.VectorSubcoreMesh(core_axis_name="c", subcore_axis_name="s",
                                num_cores=1),
    compiler_params=pltpu.CompilerParams(needs_layout_passes=False),
    scratch_types=[
        pltpu.VMEM((NHEADS * NUM_PATCHES,), jnp.float32),
        pltpu.VMEM((NUM_PATCHES,), jnp.float32),
        pltpu.VMEM((NUM_PATCHES,), jnp.int32),
        pltpu.VMEM((NUM_PATCHES,), jnp.float32),
    ],
)(_sc_mask_body)


def _tc_body(mask_ref, img_ref, out_ref):
    b = pl.program_id(0)
    p_cols = jnp.where(
        jax.lax.broadcasted_iota(jnp.int32, (FEAT, IMG), 1) // PATCH
        == jax.lax.broadcasted_iota(jnp.int32, (FEAT, IMG), 0),
        1.0, 0.0).astype(jnp.float32)
    p_rows = jnp.where(
        jax.lax.broadcasted_iota(jnp.int32, (IMG, FEAT), 0) // PATCH
        == jax.lax.broadcasted_iota(jnp.int32, (IMG, FEAT), 1),
        1.0, 0.0).astype(jnp.float32)
    for j in range(BB):
        m = mask_ref[BB * b + j]
        mp = jax.lax.dot_general(m, p_cols, (((1,), (0,)), ((), ())),
                                 preferred_element_type=jnp.float32)
        m_full = jax.lax.dot_general(p_rows, mp, (((1,), (0,)), ((), ())),
                                     preferred_element_type=jnp.float32)
        out_ref[j] = img_ref[j] * m_full[None, :, :]


def kernel(x, input_images):
    B, NH = x.shape[0], x.shape[1]
    att = x[:, :, 0, 1:].reshape(B, NH * NUM_PATCHES)
    mask = _sc_mask(att)
    mask24 = mask.reshape(B, FEAT, FEAT)
    return pl.pallas_call(
        _tc_body,
        grid=(B // BB,),
        in_specs=[
            pl.BlockSpec((B, FEAT, FEAT), lambda b: (0, 0, 0)),
            pl.BlockSpec((BB, 3, IMG, IMG), lambda b: (b, 0, 0, 0)),
        ],
        out_specs=pl.BlockSpec((BB, 3, IMG, IMG), lambda b: (b, 0, 0, 0)),
        out_shape=jax.ShapeDtypeStruct(input_images.shape, input_images.dtype),
    )(mask24, input_images)
